# baseline (speedup 1.0000x reference)
import functools

import jax
import jax.numpy as jnp
from jax import lax
from jax.experimental import pallas as pl
from jax.experimental.pallas import tpu as pltpu
from jax.experimental.pallas import tpu_sc as plsc

_B = 16384
_D = 32
_L = 16
_NW = 32
_BPW = _B // _NW
_CH = 128
_NCH = _BPW // _CH

_mesh = plsc.VectorSubcoreMesh(core_axis_name="c", subcore_axis_name="s")


@functools.partial(
    pl.kernel,
    out_type=jax.ShapeDtypeStruct((_B,), jnp.float32),
    mesh=_mesh,
    scratch_types=[
        pltpu.VMEM((_NCH, _CH), jnp.int32),
        pltpu.VMEM((_NCH, _CH), jnp.int32),
        pltpu.VMEM((_BPW, _D), jnp.float32),
        pltpu.VMEM((_BPW, _D), jnp.float32),
        pltpu.VMEM((_BPW,), jnp.float32),
        pltpu.VMEM((3 * _L,), jnp.float32),
        pltpu.SemaphoreType.DMA,
    ],
    compiler_params=pltpu.CompilerParams(
        needs_layout_passes=False, use_tc_tiling_on_sc=False),
)
def _gmf_sc(user_hbm, item_hbm, utab_hbm, itab_hbm, params_hbm, out_hbm,
            idx_u, idx_i, rows_u, rows_i, out_v, p_v, sem):
    wid = lax.axis_index("s") * 2 + lax.axis_index("c")
    base = wid * _BPW

    pltpu.sync_copy(params_hbm, p_v)
    for j in range(_NCH):
        pltpu.sync_copy(user_hbm.at[pl.ds(base + j * _CH, _CH)], idx_u.at[j])
        pltpu.sync_copy(item_hbm.at[pl.ds(base + j * _CH, _CH)], idx_i.at[j])

    copies = []
    for j in range(_NCH):
        copies.append(pltpu.async_copy(
            utab_hbm.at[idx_u.at[j]], rows_u.at[pl.ds(j * _CH, _CH)], sem))
        copies.append(pltpu.async_copy(
            itab_hbm.at[idx_i.at[j]], rows_i.at[pl.ds(j * _CH, _CH)], sem))
    for cp in copies:
        cp.wait()

    w_lo = p_v[pl.ds(0, _L)]
    w_hi = p_v[pl.ds(_L, _L)]
    bias = p_v[pl.ds(2 * _L, _L)][0]
    lane = lax.iota(jnp.int32, _L)
    last_lane = lane == (_L - 1)

    def body(g, carry):
        u_lo = rows_u[g, pl.ds(0, _L)]
        u_hi = rows_u[g, pl.ds(_L, _L)]
        i_lo = rows_i[g, pl.ds(0, _L)]
        i_hi = rows_i[g, pl.ds(_L, _L)]
        sv = u_lo * i_lo * w_lo + u_hi * i_hi * w_hi
        total = plsc.cumsum(sv) + bias
        plsc.store_scatter(out_v, [jnp.broadcast_to(g, (_L,))], total,
                           mask=last_lane)
        return carry

    lax.fori_loop(0, _BPW, body, 0)

    pltpu.sync_copy(out_v, out_hbm.at[pl.ds(base, _BPW)])


def kernel(user, item, user_table, item_table, W, b):
    params = jnp.concatenate(
        [W.reshape(-1), b.reshape(-1), jnp.zeros((15,), jnp.float32)])
    return _gmf_sc(user.astype(jnp.int32), item.astype(jnp.int32),
                   user_table, item_table, params)


# speedup vs baseline: 3.8542x; 3.8542x over previous
import functools

import jax
import jax.numpy as jnp
from jax import lax
from jax.experimental import pallas as pl
from jax.experimental.pallas import tpu as pltpu
from jax.experimental.pallas import tpu_sc as plsc

_B = 16384
_D = 32
_L = 16
_NW = 32
_BPW = _B // _NW
_NB = _BPW // _L
_RING = 16

_mesh = plsc.VectorSubcoreMesh(core_axis_name="c", subcore_axis_name="s")


@functools.partial(
    pl.kernel,
    out_type=jax.ShapeDtypeStruct((_B,), jnp.float32),
    mesh=_mesh,
    scratch_types=[
        pltpu.VMEM((_BPW,), jnp.int32),
        pltpu.VMEM((_BPW,), jnp.int32),
        pltpu.VMEM((_RING, _D, 128), jnp.float32),
        pltpu.VMEM((_BPW * _D,), jnp.float32),
        pltpu.VMEM((_BPW,), jnp.float32),
        pltpu.VMEM((3 * _L,), jnp.float32),
        pltpu.SemaphoreType.DMA((_RING,)),
    ],
    compiler_params=pltpu.CompilerParams(needs_layout_passes=False),
)
def _gmf_sc(user_hbm, item_hbm, utab_hbm, itab_hbm, params_hbm, out_hbm,
            idx_u, idx_i, blk, part_v, out_v, p_v, sems):
    wid = lax.axis_index("s") * 2 + lax.axis_index("c")
    base = wid * _BPW

    pltpu.sync_copy(params_hbm, p_v)
    pltpu.sync_copy(user_hbm.at[pl.ds(base, _BPW)], idx_u)
    pltpu.sync_copy(item_hbm.at[pl.ds(base, _BPW)], idx_i)

    w_lo = p_v[pl.ds(0, _L)]
    w_hi = p_v[pl.ds(_L, _L)]
    bias = p_v[pl.ds(2 * _L, _L)][0]
    dio = lax.iota(jnp.int32, _L)
    dio_hi = dio + _L
    last_lane = dio == (_L - 1)

    def fire(tab, cv, j):
        off = pl.multiple_of(cv[j], 128)
        pltpu.async_copy(tab.at[:, pl.ds(off, 128)], blk.at[j], sems.at[j])

    def drain(tab, j):
        pltpu.make_async_copy(
            tab.at[:, pl.ds(0, 128)], blk.at[j], sems.at[j]).wait()

    def run_phase(tab, idx_ref, emit):
        iv0 = idx_ref[pl.ds(0, _L)]
        cv0 = iv0 & (-128)
        for j in range(_RING):
            fire(tab, cv0, j)

        def body(g, carry):
            goff = pl.multiple_of(g * _L, _L)
            iv = idx_ref[pl.ds(goff, _L)]
            lv = iv & 127
            gn = jnp.minimum(g + 1, _NB - 1) * _L
            ivn = idx_ref[pl.ds(pl.multiple_of(gn, _L), _L)]
            cvn = ivn & (-128)
            bb = g * _L
            for j in range(_RING):
                drain(tab, j)
                lane = jnp.broadcast_to(lv[j], (_L,))
                vlo = plsc.load_gather(blk.at[j], [dio, lane])
                vhi = plsc.load_gather(blk.at[j], [dio_hi, lane])
                emit(bb + j, vlo, vhi)

                @pl.when(g < _NB - 1)
                def _():
                    fire(tab, cvn, j)
            return carry

        lax.fori_loop(0, _NB, body, 0)

    def emit_user(b_local, vlo, vhi):
        po = pl.multiple_of(b_local * _D, _D)
        part_v[pl.ds(po, _L)] = vlo * w_lo
        part_v[pl.ds(po + _L, _L)] = vhi * w_hi

    def emit_item(b_local, vlo, vhi):
        po = pl.multiple_of(b_local * _D, _D)
        plo = part_v[pl.ds(po, _L)]
        phi = part_v[pl.ds(po + _L, _L)]
        sv = vlo * plo + vhi * phi
        total = plsc.cumsum(sv) + bias
        plsc.store_scatter(out_v, [jnp.broadcast_to(b_local, (_L,))], total,
                           mask=last_lane)

    run_phase(utab_hbm, idx_u, emit_user)
    run_phase(itab_hbm, idx_i, emit_item)

    pltpu.sync_copy(out_v, out_hbm.at[pl.ds(base, _BPW)])


def kernel(user, item, user_table, item_table, W, b):
    params = jnp.concatenate(
        [W.reshape(-1), b.reshape(-1), jnp.zeros((15,), jnp.float32)])
    return _gmf_sc(user.astype(jnp.int32), item.astype(jnp.int32),
                   user_table.T, item_table.T, params)
